# baseline (speedup 1.0000x reference)
import functools

import jax
import jax.numpy as jnp
from jax import lax
from jax.experimental import pallas as pl
from jax.experimental.pallas import tpu as pltpu
from jax.experimental.pallas import tpu_sc as plsc

N = 10000
E = 320000
IN_DIM = 128
OUT_DIM = 64

NC = 2
NS = 16
LANES = 16
NW = NC * NS

CHUNK = 128
CPT = 80
E_PAD = NW * CPT * CHUNK
ROWS_PT = 640
N_ACC = NS * ROWS_PT


def _mesh():
    return plsc.VectorSubcoreMesh(
        core_axis_name="c", subcore_axis_name="s",
        num_cores=NC, num_subcores=NS)


_SC_PARAMS = pltpu.CompilerParams(
    needs_layout_passes=False, use_tc_tiling_on_sc=False)


def _sc_gat_body(h_hbm, asrc_hbm, adst_hbm, src_hbm, dst_hbm,
                 num_out, den_out,
                 src_v, dst_v, rows_a, rows_b, as_a, as_b, ad_a, ad_b,
                 exv, zrow_v, zden_v, num_sh, den_sh, sem_a, sem_b):
    c = lax.axis_index("c")
    s = lax.axis_index("s")
    wid = c * NS + s
    zero16 = jnp.zeros((LANES,), jnp.float32)
    for r in range(CHUNK):
        for d in range(OUT_DIM // LANES):
            zrow_v[r, pl.ds(d * LANES, LANES)] = zero16
    for i in range(CHUNK // LANES):
        zden_v[pl.ds(i * LANES, LANES)] = zero16
    base = s * ROWS_PT
    for k in range(ROWS_PT // CHUNK):
        pltpu.sync_copy(zrow_v, num_sh.at[pl.ds(base + k * CHUNK, CHUNK)])
        pltpu.sync_copy(zden_v, den_sh.at[pl.ds(base + k * CHUNK, CHUNK)])
    plsc.subcore_barrier()

    pltpu.sync_copy(src_hbm.at[wid], src_v)
    pltpu.sync_copy(dst_hbm.at[wid], dst_v)

    def start(j, rows_t, as_t, ad_t, sem):
        return [
            pltpu.async_copy(asrc_hbm.at[src_v.at[j]], as_t, sem),
            pltpu.async_copy(adst_hbm.at[dst_v.at[j]], ad_t, sem),
            pltpu.async_copy(h_hbm.at[src_v.at[j]], rows_t, sem),
        ]

    def process(j, rows_t, as_t, ad_t):
        didx = dst_v.at[j]
        for i in range(CHUNK // LANES):
            sl = pl.ds(i * LANES, LANES)
            t = as_t[sl] + ad_t[sl]
            t = jnp.where(t >= 0.0, t, 0.2 * t)
            exv[sl] = jnp.exp(t)
        zero = jnp.zeros((LANES,), jnp.int32)

        @pl.loop(0, CHUNK, unroll=4)
        def _scale(e):
            b = plsc.load_gather(exv, [zero + e])
            for d in range(OUT_DIM // LANES):
                sl = pl.ds(d * LANES, LANES)
                rows_t[e, sl] = rows_t[e, sl] * b

        pltpu.sync_copy(exv, den_sh.at[didx], add=True)
        pltpu.sync_copy(rows_t, num_sh.at[didx], add=True)

    @pl.loop(0, CPT // 2)
    def _pair(p):
        j0 = p * 2
        j1 = j0 + 1
        c0 = start(j0, rows_a, as_a, ad_a, sem_a)
        c1 = start(j1, rows_b, as_b, ad_b, sem_b)
        for cp in c0:
            cp.wait()
        process(j0, rows_a, as_a, ad_a)
        for cp in c1:
            cp.wait()
        process(j1, rows_b, as_b, ad_b)

    plsc.subcore_barrier()
    pltpu.sync_copy(num_sh.at[pl.ds(base, ROWS_PT)],
                    num_out.at[c, pl.ds(base, ROWS_PT)])
    pltpu.sync_copy(den_sh.at[pl.ds(base, ROWS_PT)],
                    den_out.at[c, pl.ds(base, ROWS_PT)])


def _sc_gat(h, asrc, adst, srcp, dstp):
    f = pl.kernel(
        _sc_gat_body,
        out_type=(jax.ShapeDtypeStruct((NC, N_ACC, OUT_DIM), jnp.float32),
                  jax.ShapeDtypeStruct((NC, N_ACC), jnp.float32)),
        mesh=_mesh(),
        compiler_params=_SC_PARAMS,
        scratch_types=[
            pltpu.VMEM((CPT, CHUNK), jnp.int32),
            pltpu.VMEM((CPT, CHUNK), jnp.int32),
            pltpu.VMEM((CHUNK, OUT_DIM), jnp.float32),
            pltpu.VMEM((CHUNK, OUT_DIM), jnp.float32),
            pltpu.VMEM((CHUNK,), jnp.float32),
            pltpu.VMEM((CHUNK,), jnp.float32),
            pltpu.VMEM((CHUNK,), jnp.float32),
            pltpu.VMEM((CHUNK,), jnp.float32),
            pltpu.VMEM((CHUNK,), jnp.float32),
            pltpu.VMEM((CHUNK, OUT_DIM), jnp.float32),
            pltpu.VMEM((CHUNK,), jnp.float32),
            pltpu.VMEM_SHARED((N_ACC, OUT_DIM), jnp.float32),
            pltpu.VMEM_SHARED((N_ACC,), jnp.float32),
            pltpu.SemaphoreType.DMA,
            pltpu.SemaphoreType.DMA,
        ],
    )
    return f(h, asrc, adst, srcp, dstp)


def _sc_agg_body(h1_hbm, src_hbm, dst_hbm, g_out,
                 src_v, dst_v, r0, r1, zrow_v, g_sh, s0, s1):
    c = lax.axis_index("c")
    s = lax.axis_index("s")
    wid = c * NS + s
    zero16 = jnp.zeros((LANES,), jnp.float32)
    for r in range(CHUNK):
        for d in range(OUT_DIM // LANES):
            zrow_v[r, pl.ds(d * LANES, LANES)] = zero16
    base = s * ROWS_PT
    for k in range(ROWS_PT // CHUNK):
        pltpu.sync_copy(zrow_v, g_sh.at[pl.ds(base + k * CHUNK, CHUNK)])
    plsc.subcore_barrier()

    pltpu.sync_copy(src_hbm.at[wid], src_v)
    pltpu.sync_copy(dst_hbm.at[wid], dst_v)

    @pl.loop(0, CPT)
    def _chunk(j):
        pltpu.sync_copy(h1_hbm.at[src_v.at[j]], r0, )
        pltpu.sync_copy(r0, g_sh.at[dst_v.at[j]], add=True)

    plsc.subcore_barrier()
    pltpu.sync_copy(g_sh.at[pl.ds(base, ROWS_PT)],
                    g_out.at[c, pl.ds(base, ROWS_PT)])


def _sc_agg(h1, srcp, dstp):
    f = pl.kernel(
        _sc_agg_body,
        out_type=jax.ShapeDtypeStruct((NC, N_ACC, OUT_DIM), jnp.float32),
        mesh=_mesh(),
        compiler_params=_SC_PARAMS,
        scratch_types=[
            pltpu.VMEM((CPT, CHUNK), jnp.int32),
            pltpu.VMEM((CPT, CHUNK), jnp.int32),
            pltpu.VMEM((CHUNK, OUT_DIM), jnp.float32),
            pltpu.VMEM((CHUNK, OUT_DIM), jnp.float32),
            pltpu.VMEM((CHUNK, OUT_DIM), jnp.float32),
            pltpu.VMEM_SHARED((N_ACC, OUT_DIM), jnp.float32),
            pltpu.SemaphoreType.DMA,
            pltpu.SemaphoreType.DMA,
        ],
    )
    return f(h1, srcp, dstp)


_BLK = 2000


def _tc_proj_body(x_ref, w_ref, as_ref, ad_ref, h_ref, asrc_ref, adst_ref):
    h = jnp.dot(x_ref[...], w_ref[...], preferred_element_type=jnp.float32)
    h_ref[...] = h
    asrc_ref[...] = jnp.dot(h, as_ref[...], preferred_element_type=jnp.float32)
    adst_ref[...] = jnp.dot(h, ad_ref[...], preferred_element_type=jnp.float32)


def _tc_proj(x, w1, att_s, att_d):
    return pl.pallas_call(
        _tc_proj_body,
        grid=(N // _BLK,),
        in_specs=[
            pl.BlockSpec((_BLK, IN_DIM), lambda i: (i, 0)),
            pl.BlockSpec((IN_DIM, OUT_DIM), lambda i: (0, 0)),
            pl.BlockSpec((OUT_DIM, 1), lambda i: (0, 0)),
            pl.BlockSpec((OUT_DIM, 1), lambda i: (0, 0)),
        ],
        out_specs=[
            pl.BlockSpec((_BLK, OUT_DIM), lambda i: (i, 0)),
            pl.BlockSpec((_BLK, 1), lambda i: (i, 0)),
            pl.BlockSpec((_BLK, 1), lambda i: (i, 0)),
        ],
        out_shape=[
            jax.ShapeDtypeStruct((N, OUT_DIM), jnp.float32),
            jax.ShapeDtypeStruct((N, 1), jnp.float32),
            jax.ShapeDtypeStruct((N, 1), jnp.float32),
        ],
    )(x, w1, att_s, att_d)


def _tc_combine_body(num_ref, den_ref, h1_ref):
    n = num_ref[0] + num_ref[1]
    d = den_ref[0] + den_ref[1] + 1e-16
    o = n / d
    h1_ref[...] = jnp.where(o > 0.0, o, jnp.exp(jnp.minimum(o, 0.0)) - 1.0)


def _tc_combine(num, den):
    den = den.reshape(NC, N_ACC, 1)
    return pl.pallas_call(
        _tc_combine_body,
        grid=(N // _BLK,),
        in_specs=[
            pl.BlockSpec((NC, _BLK, OUT_DIM), lambda i: (0, i, 0)),
            pl.BlockSpec((NC, _BLK, 1), lambda i: (0, i, 0)),
        ],
        out_specs=pl.BlockSpec((_BLK, OUT_DIM), lambda i: (i, 0)),
        out_shape=jax.ShapeDtypeStruct((N, OUT_DIM), jnp.float32),
    )(num, den)


def _tc_out_body(g_ref, w_ref, h4_ref):
    g = g_ref[0] + g_ref[1]
    h4_ref[...] = jnp.dot(g, w_ref[...], preferred_element_type=jnp.float32)


def _tc_out(g, w4):
    return pl.pallas_call(
        _tc_out_body,
        grid=(N // _BLK,),
        in_specs=[
            pl.BlockSpec((NC, _BLK, OUT_DIM), lambda i: (0, i, 0)),
            pl.BlockSpec((OUT_DIM, IN_DIM), lambda i: (0, 0)),
        ],
        out_specs=pl.BlockSpec((_BLK, IN_DIM), lambda i: (i, 0)),
        out_shape=jax.ShapeDtypeStruct((N, IN_DIM), jnp.float32),
    )(g, w4)


def kernel(features, adjs, W1, att_src1, att_dst1, W4):
    src = adjs[0]
    dst = adjs[1]
    pad = E_PAD - E
    srcp = jnp.concatenate(
        [src, jnp.zeros((pad,), jnp.int32)]).reshape(NW, CPT, CHUNK)
    dstp = jnp.concatenate(
        [dst, jnp.full((pad,), N, jnp.int32)]).reshape(NW, CPT, CHUNK)

    h, asrc, adst = _tc_proj(features, W1,
                             att_src1.reshape(OUT_DIM, 1),
                             att_dst1.reshape(OUT_DIM, 1))
    num, den = _sc_gat(h, asrc.reshape(N), adst.reshape(N), srcp, dstp)
    h1 = _tc_combine(num, den)
    g = _sc_agg(h1, srcp, dstp)
    h4 = _tc_out(g, W4)
    return (h1, h4)


# speedup vs baseline: 1.0751x; 1.0751x over previous
import functools

import jax
import jax.numpy as jnp
from jax import lax
from jax.experimental import pallas as pl
from jax.experimental.pallas import tpu as pltpu
from jax.experimental.pallas import tpu_sc as plsc

N = 10000
E = 320000
IN_DIM = 128
OUT_DIM = 64

NC = 2
NS = 16
LANES = 16
NW = NC * NS

CHUNK = 128
CPT = (E + NW * CHUNK - 1) // (NW * CHUNK)
E_PAD = NW * CPT * CHUNK
ROWS_PT = 640
N_ACC = NS * ROWS_PT


def _mesh():
    return plsc.VectorSubcoreMesh(
        core_axis_name="c", subcore_axis_name="s",
        num_cores=NC, num_subcores=NS)


_SC_PARAMS = pltpu.CompilerParams(
    needs_layout_passes=False, use_tc_tiling_on_sc=False)


def _sc_gat_body(h_hbm, asrc_hbm, adst_hbm, src_hbm, dst_hbm,
                 num_out, den_out,
                 src_v, dst_v, rows_a, rows_b, as_a, as_b, ad_a, ad_b,
                 exv, zrow_v, zden_v, num_sh, den_sh, sem_a, sem_b):
    c = lax.axis_index("c")
    s = lax.axis_index("s")
    wid = c * NS + s
    zero16 = jnp.zeros((LANES,), jnp.float32)
    for r in range(CHUNK):
        for d in range(OUT_DIM // LANES):
            zrow_v[r, pl.ds(d * LANES, LANES)] = zero16
    for i in range(CHUNK // LANES):
        zden_v[pl.ds(i * LANES, LANES)] = zero16
    base = s * ROWS_PT
    for k in range(ROWS_PT // CHUNK):
        pltpu.sync_copy(zrow_v, num_sh.at[pl.ds(base + k * CHUNK, CHUNK)])
        pltpu.sync_copy(zden_v, den_sh.at[pl.ds(base + k * CHUNK, CHUNK)])
    plsc.subcore_barrier()

    pltpu.sync_copy(src_hbm.at[wid], src_v)
    pltpu.sync_copy(dst_hbm.at[wid], dst_v)

    @pl.loop(0, CPT)
    def _chunk(j):
        sidx = src_v.at[j]
        didx = dst_v.at[j]
        pltpu.sync_copy(asrc_hbm.at[sidx], as_a)
        pltpu.sync_copy(adst_hbm.at[didx], ad_a)
        pltpu.sync_copy(h_hbm.at[sidx], rows_a)
        for i in range(CHUNK // LANES):
            sl = pl.ds(i * LANES, LANES)
            t = as_a[sl] + ad_a[sl]
            t = jnp.where(t >= 0.0, t, 0.2 * t)
            exv[sl] = jnp.exp(t)
        zero = jnp.zeros((LANES,), jnp.int32)

        @pl.loop(0, CHUNK)
        def _scale(e):
            b = plsc.load_gather(exv, [zero + e])
            for d in range(OUT_DIM // LANES):
                sl = pl.ds(d * LANES, LANES)
                rows_a[e, sl] = rows_a[e, sl] * b

        pltpu.sync_copy(exv, den_sh.at[didx], add=True)
        pltpu.sync_copy(rows_a, num_sh.at[didx], add=True)

    plsc.subcore_barrier()
    pltpu.sync_copy(num_sh.at[pl.ds(base, ROWS_PT)],
                    num_out.at[c, pl.ds(base, ROWS_PT)])
    pltpu.sync_copy(den_sh.at[pl.ds(base, ROWS_PT)],
                    den_out.at[c, pl.ds(base, ROWS_PT)])


def _sc_gat(h, asrc, adst, srcp, dstp):
    f = pl.kernel(
        _sc_gat_body,
        out_type=(jax.ShapeDtypeStruct((NC, N_ACC, OUT_DIM), jnp.float32),
                  jax.ShapeDtypeStruct((NC, N_ACC), jnp.float32)),
        mesh=_mesh(),
        compiler_params=_SC_PARAMS,
        scratch_types=[
            pltpu.VMEM((CPT, CHUNK), jnp.int32),
            pltpu.VMEM((CPT, CHUNK), jnp.int32),
            pltpu.VMEM((CHUNK, OUT_DIM), jnp.float32),
            pltpu.VMEM((CHUNK, OUT_DIM), jnp.float32),
            pltpu.VMEM((CHUNK,), jnp.float32),
            pltpu.VMEM((CHUNK,), jnp.float32),
            pltpu.VMEM((CHUNK,), jnp.float32),
            pltpu.VMEM((CHUNK,), jnp.float32),
            pltpu.VMEM((CHUNK,), jnp.float32),
            pltpu.VMEM((CHUNK, OUT_DIM), jnp.float32),
            pltpu.VMEM((CHUNK,), jnp.float32),
            pltpu.VMEM_SHARED((N_ACC, OUT_DIM), jnp.float32),
            pltpu.VMEM_SHARED((N_ACC,), jnp.float32),
            pltpu.SemaphoreType.DMA,
            pltpu.SemaphoreType.DMA,
        ],
    )
    return f(h, asrc, adst, srcp, dstp)


def _sc_agg_body(h1_hbm, src_hbm, dst_hbm, g_out,
                 src_v, dst_v, r0, r1, zrow_v, g_sh, s0, s1):
    c = lax.axis_index("c")
    s = lax.axis_index("s")
    wid = c * NS + s
    zero16 = jnp.zeros((LANES,), jnp.float32)
    for r in range(CHUNK):
        for d in range(OUT_DIM // LANES):
            zrow_v[r, pl.ds(d * LANES, LANES)] = zero16
    base = s * ROWS_PT
    for k in range(ROWS_PT // CHUNK):
        pltpu.sync_copy(zrow_v, g_sh.at[pl.ds(base + k * CHUNK, CHUNK)])
    plsc.subcore_barrier()

    pltpu.sync_copy(src_hbm.at[wid], src_v)
    pltpu.sync_copy(dst_hbm.at[wid], dst_v)

    @pl.loop(0, CPT)
    def _chunk(j):
        pltpu.sync_copy(h1_hbm.at[src_v.at[j]], r0, )
        pltpu.sync_copy(r0, g_sh.at[dst_v.at[j]], add=True)

    plsc.subcore_barrier()
    pltpu.sync_copy(g_sh.at[pl.ds(base, ROWS_PT)],
                    g_out.at[c, pl.ds(base, ROWS_PT)])


def _sc_agg(h1, srcp, dstp):
    f = pl.kernel(
        _sc_agg_body,
        out_type=jax.ShapeDtypeStruct((NC, N_ACC, OUT_DIM), jnp.float32),
        mesh=_mesh(),
        compiler_params=_SC_PARAMS,
        scratch_types=[
            pltpu.VMEM((CPT, CHUNK), jnp.int32),
            pltpu.VMEM((CPT, CHUNK), jnp.int32),
            pltpu.VMEM((CHUNK, OUT_DIM), jnp.float32),
            pltpu.VMEM((CHUNK, OUT_DIM), jnp.float32),
            pltpu.VMEM((CHUNK, OUT_DIM), jnp.float32),
            pltpu.VMEM_SHARED((N_ACC, OUT_DIM), jnp.float32),
            pltpu.SemaphoreType.DMA,
            pltpu.SemaphoreType.DMA,
        ],
    )
    return f(h1, srcp, dstp)


_BLK = 2000


def _tc_proj_body(x_ref, w_ref, as_ref, ad_ref, h_ref, asrc_ref, adst_ref):
    h = jnp.dot(x_ref[...], w_ref[...], preferred_element_type=jnp.float32)
    h_ref[...] = h
    asrc_ref[...] = jnp.dot(h, as_ref[...], preferred_element_type=jnp.float32)
    adst_ref[...] = jnp.dot(h, ad_ref[...], preferred_element_type=jnp.float32)


def _tc_proj(x, w1, att_s, att_d):
    return pl.pallas_call(
        _tc_proj_body,
        grid=(N // _BLK,),
        in_specs=[
            pl.BlockSpec((_BLK, IN_DIM), lambda i: (i, 0)),
            pl.BlockSpec((IN_DIM, OUT_DIM), lambda i: (0, 0)),
            pl.BlockSpec((OUT_DIM, 1), lambda i: (0, 0)),
            pl.BlockSpec((OUT_DIM, 1), lambda i: (0, 0)),
        ],
        out_specs=[
            pl.BlockSpec((_BLK, OUT_DIM), lambda i: (i, 0)),
            pl.BlockSpec((_BLK, 1), lambda i: (i, 0)),
            pl.BlockSpec((_BLK, 1), lambda i: (i, 0)),
        ],
        out_shape=[
            jax.ShapeDtypeStruct((N, OUT_DIM), jnp.float32),
            jax.ShapeDtypeStruct((N, 1), jnp.float32),
            jax.ShapeDtypeStruct((N, 1), jnp.float32),
        ],
    )(x, w1, att_s, att_d)


def _tc_combine_body(num_ref, den_ref, h1_ref):
    n = num_ref[0] + num_ref[1]
    d = den_ref[0] + den_ref[1] + 1e-16
    o = n / d
    h1_ref[...] = jnp.where(o > 0.0, o, jnp.exp(jnp.minimum(o, 0.0)) - 1.0)


def _tc_combine(num, den):
    den = den.reshape(NC, N_ACC, 1)
    return pl.pallas_call(
        _tc_combine_body,
        grid=(N // _BLK,),
        in_specs=[
            pl.BlockSpec((NC, _BLK, OUT_DIM), lambda i: (0, i, 0)),
            pl.BlockSpec((NC, _BLK, 1), lambda i: (0, i, 0)),
        ],
        out_specs=pl.BlockSpec((_BLK, OUT_DIM), lambda i: (i, 0)),
        out_shape=jax.ShapeDtypeStruct((N, OUT_DIM), jnp.float32),
    )(num, den)


def _tc_out_body(g_ref, w_ref, h4_ref):
    g = g_ref[0] + g_ref[1]
    h4_ref[...] = jnp.dot(g, w_ref[...], preferred_element_type=jnp.float32)


def _tc_out(g, w4):
    return pl.pallas_call(
        _tc_out_body,
        grid=(N // _BLK,),
        in_specs=[
            pl.BlockSpec((NC, _BLK, OUT_DIM), lambda i: (0, i, 0)),
            pl.BlockSpec((OUT_DIM, IN_DIM), lambda i: (0, 0)),
        ],
        out_specs=pl.BlockSpec((_BLK, IN_DIM), lambda i: (i, 0)),
        out_shape=jax.ShapeDtypeStruct((N, IN_DIM), jnp.float32),
    )(g, w4)


def kernel(features, adjs, W1, att_src1, att_dst1, W4):
    src = adjs[0]
    dst = adjs[1]
    pad = E_PAD - E
    srcp = jnp.concatenate(
        [src, jnp.zeros((pad,), jnp.int32)]).reshape(NW, CPT, CHUNK)
    dstp = jnp.concatenate(
        [dst, jnp.full((pad,), N, jnp.int32)]).reshape(NW, CPT, CHUNK)

    h, asrc, adst = _tc_proj(features, W1,
                             att_src1.reshape(OUT_DIM, 1),
                             att_dst1.reshape(OUT_DIM, 1))
    num, den = _sc_gat(h, asrc.reshape(N), adst.reshape(N), srcp, dstp)
    h1 = _tc_combine(num, den)
    g = _sc_agg(h1, srcp, dstp)
    h4 = _tc_out(g, W4)
    return (h1, h4)


# speedup vs baseline: 1.7347x; 1.6135x over previous
import jax
import jax.numpy as jnp
from jax import lax
from jax.experimental import pallas as pl
from jax.experimental.pallas import tpu as pltpu
from jax.experimental.pallas import tpu_sc as plsc

N = 10000
E = 320000
IN_DIM = 128
OUT_DIM = 64

NC = 2
NS = 16
LANES = 16
NW = NC * NS

CHUNK = 128
CPT = (E + NW * CHUNK - 1) // (NW * CHUNK)
E_PAD = NW * CPT * CHUNK
ROWS_PT = 640
N_ACC = NS * ROWS_PT


def _mesh():
    return plsc.VectorSubcoreMesh(
        core_axis_name="c", subcore_axis_name="s",
        num_cores=NC, num_subcores=NS)


_SC_PARAMS = pltpu.CompilerParams(
    needs_layout_passes=False, use_tc_tiling_on_sc=False)


def _sc_gat_body(h_hbm, asrc_hbm, adst_hbm, src_hbm, dst_hbm,
                 num_out, den_out,
                 src_v, dst_v, rows_a, as_a, ad_a, exv, zrow_v, zden_v,
                 h_sh, asrc_sh, adst_sh, num_sh, den_sh):
    c = lax.axis_index("c")
    s = lax.axis_index("s")
    wid = c * NS + s
    zero16 = jnp.zeros((LANES,), jnp.float32)
    for r in range(CHUNK):
        for d in range(OUT_DIM // LANES):
            zrow_v[r, pl.ds(d * LANES, LANES)] = zero16
    for i in range(CHUNK // LANES):
        zden_v[pl.ds(i * LANES, LANES)] = zero16
    base = s * ROWS_PT
    for k in range(ROWS_PT // CHUNK):
        pltpu.sync_copy(zrow_v, num_sh.at[pl.ds(base + k * CHUNK, CHUNK)])
        pltpu.sync_copy(zden_v, den_sh.at[pl.ds(base + k * CHUNK, CHUNK)])
    pltpu.sync_copy(h_hbm.at[pl.ds(base, ROWS_PT)],
                    h_sh.at[pl.ds(base, ROWS_PT)])
    pltpu.sync_copy(asrc_hbm.at[pl.ds(base, ROWS_PT)],
                    asrc_sh.at[pl.ds(base, ROWS_PT)])
    pltpu.sync_copy(adst_hbm.at[pl.ds(base, ROWS_PT)],
                    adst_sh.at[pl.ds(base, ROWS_PT)])
    plsc.subcore_barrier()

    pltpu.sync_copy(src_hbm.at[wid], src_v)
    pltpu.sync_copy(dst_hbm.at[wid], dst_v)

    @pl.loop(0, CPT)
    def _chunk(j):
        sidx = src_v.at[j]
        didx = dst_v.at[j]
        pltpu.sync_copy(asrc_sh.at[sidx], as_a)
        pltpu.sync_copy(adst_sh.at[didx], ad_a)
        pltpu.sync_copy(h_sh.at[sidx], rows_a)
        for i in range(CHUNK // LANES):
            sl = pl.ds(i * LANES, LANES)
            t = as_a[sl] + ad_a[sl]
            t = jnp.where(t >= 0.0, t, 0.2 * t)
            exv[sl] = jnp.exp(t)
        zero = jnp.zeros((LANES,), jnp.int32)

        @pl.loop(0, CHUNK)
        def _scale(e):
            b = plsc.load_gather(exv, [zero + e])
            for d in range(OUT_DIM // LANES):
                sl = pl.ds(d * LANES, LANES)
                rows_a[e, sl] = rows_a[e, sl] * b

        pltpu.sync_copy(exv, den_sh.at[didx], add=True)
        pltpu.sync_copy(rows_a, num_sh.at[didx], add=True)

    plsc.subcore_barrier()
    pltpu.sync_copy(num_sh.at[pl.ds(base, ROWS_PT)],
                    num_out.at[c, pl.ds(base, ROWS_PT)])
    pltpu.sync_copy(den_sh.at[pl.ds(base, ROWS_PT)],
                    den_out.at[c, pl.ds(base, ROWS_PT)])


def _sc_gat(h, asrc, adst, srcp, dstp):
    f = pl.kernel(
        _sc_gat_body,
        out_type=(jax.ShapeDtypeStruct((NC, N_ACC, OUT_DIM), jnp.float32),
                  jax.ShapeDtypeStruct((NC, N_ACC), jnp.float32)),
        mesh=_mesh(),
        compiler_params=_SC_PARAMS,
        scratch_types=[
            pltpu.VMEM((CPT, CHUNK), jnp.int32),
            pltpu.VMEM((CPT, CHUNK), jnp.int32),
            pltpu.VMEM((CHUNK, OUT_DIM), jnp.float32),
            pltpu.VMEM((CHUNK,), jnp.float32),
            pltpu.VMEM((CHUNK,), jnp.float32),
            pltpu.VMEM((CHUNK,), jnp.float32),
            pltpu.VMEM((CHUNK, OUT_DIM), jnp.float32),
            pltpu.VMEM((CHUNK,), jnp.float32),
            pltpu.VMEM_SHARED((N_ACC, OUT_DIM), jnp.float32),
            pltpu.VMEM_SHARED((N_ACC,), jnp.float32),
            pltpu.VMEM_SHARED((N_ACC,), jnp.float32),
            pltpu.VMEM_SHARED((N_ACC, OUT_DIM), jnp.float32),
            pltpu.VMEM_SHARED((N_ACC,), jnp.float32),
        ],
    )
    return f(h, asrc, adst, srcp, dstp)


def _sc_agg_body(h1_hbm, src_hbm, dst_hbm, g_out,
                 src_v, dst_v, r0, zrow_v, h1_sh, g_sh):
    c = lax.axis_index("c")
    s = lax.axis_index("s")
    wid = c * NS + s
    zero16 = jnp.zeros((LANES,), jnp.float32)
    for r in range(CHUNK):
        for d in range(OUT_DIM // LANES):
            zrow_v[r, pl.ds(d * LANES, LANES)] = zero16
    base = s * ROWS_PT
    for k in range(ROWS_PT // CHUNK):
        pltpu.sync_copy(zrow_v, g_sh.at[pl.ds(base + k * CHUNK, CHUNK)])
    pltpu.sync_copy(h1_hbm.at[pl.ds(base, ROWS_PT)],
                    h1_sh.at[pl.ds(base, ROWS_PT)])
    plsc.subcore_barrier()

    pltpu.sync_copy(src_hbm.at[wid], src_v)
    pltpu.sync_copy(dst_hbm.at[wid], dst_v)

    @pl.loop(0, CPT)
    def _chunk(j):
        pltpu.sync_copy(h1_sh.at[src_v.at[j]], r0)
        pltpu.sync_copy(r0, g_sh.at[dst_v.at[j]], add=True)

    plsc.subcore_barrier()
    pltpu.sync_copy(g_sh.at[pl.ds(base, ROWS_PT)],
                    g_out.at[c, pl.ds(base, ROWS_PT)])


def _sc_agg(h1, srcp, dstp):
    f = pl.kernel(
        _sc_agg_body,
        out_type=jax.ShapeDtypeStruct((NC, N_ACC, OUT_DIM), jnp.float32),
        mesh=_mesh(),
        compiler_params=_SC_PARAMS,
        scratch_types=[
            pltpu.VMEM((CPT, CHUNK), jnp.int32),
            pltpu.VMEM((CPT, CHUNK), jnp.int32),
            pltpu.VMEM((CHUNK, OUT_DIM), jnp.float32),
            pltpu.VMEM((CHUNK, OUT_DIM), jnp.float32),
            pltpu.VMEM_SHARED((N_ACC, OUT_DIM), jnp.float32),
            pltpu.VMEM_SHARED((N_ACC, OUT_DIM), jnp.float32),
        ],
    )
    return f(h1, srcp, dstp)


_BLK = 2048


def _tc_proj_body(x_ref, w_ref, as_ref, ad_ref, h_ref, asrc_ref, adst_ref):
    h = jnp.dot(x_ref[...], w_ref[...], preferred_element_type=jnp.float32)
    h_ref[...] = h
    asrc_ref[...] = jnp.dot(h, as_ref[...], preferred_element_type=jnp.float32)
    adst_ref[...] = jnp.dot(h, ad_ref[...], preferred_element_type=jnp.float32)


def _tc_proj(x, w1, att_s, att_d):
    return pl.pallas_call(
        _tc_proj_body,
        grid=(N_ACC // _BLK,),
        in_specs=[
            pl.BlockSpec((_BLK, IN_DIM), lambda i: (i, 0)),
            pl.BlockSpec((IN_DIM, OUT_DIM), lambda i: (0, 0)),
            pl.BlockSpec((OUT_DIM, 1), lambda i: (0, 0)),
            pl.BlockSpec((OUT_DIM, 1), lambda i: (0, 0)),
        ],
        out_specs=[
            pl.BlockSpec((_BLK, OUT_DIM), lambda i: (i, 0)),
            pl.BlockSpec((_BLK, 1), lambda i: (i, 0)),
            pl.BlockSpec((_BLK, 1), lambda i: (i, 0)),
        ],
        out_shape=[
            jax.ShapeDtypeStruct((N_ACC, OUT_DIM), jnp.float32),
            jax.ShapeDtypeStruct((N_ACC, 1), jnp.float32),
            jax.ShapeDtypeStruct((N_ACC, 1), jnp.float32),
        ],
    )(x, w1, att_s, att_d)


def _tc_combine_body(num_ref, den_ref, h1_ref):
    n = num_ref[0] + num_ref[1]
    d = den_ref[0] + den_ref[1] + 1e-16
    o = n / d
    h1_ref[...] = jnp.where(o > 0.0, o, jnp.exp(jnp.minimum(o, 0.0)) - 1.0)


def _tc_combine(num, den):
    den = den.reshape(NC, N_ACC, 1)
    return pl.pallas_call(
        _tc_combine_body,
        grid=(N_ACC // _BLK,),
        in_specs=[
            pl.BlockSpec((NC, _BLK, OUT_DIM), lambda i: (0, i, 0)),
            pl.BlockSpec((NC, _BLK, 1), lambda i: (0, i, 0)),
        ],
        out_specs=pl.BlockSpec((_BLK, OUT_DIM), lambda i: (i, 0)),
        out_shape=jax.ShapeDtypeStruct((N_ACC, OUT_DIM), jnp.float32),
    )(num, den)


def _tc_out_body(g_ref, w_ref, h4_ref):
    g = g_ref[0] + g_ref[1]
    h4_ref[...] = jnp.dot(g, w_ref[...], preferred_element_type=jnp.float32)


def _tc_out(g, w4):
    return pl.pallas_call(
        _tc_out_body,
        grid=(N // 2000,),
        in_specs=[
            pl.BlockSpec((NC, 2000, OUT_DIM), lambda i: (0, i, 0)),
            pl.BlockSpec((OUT_DIM, IN_DIM), lambda i: (0, 0)),
        ],
        out_specs=pl.BlockSpec((2000, IN_DIM), lambda i: (i, 0)),
        out_shape=jax.ShapeDtypeStruct((N, IN_DIM), jnp.float32),
    )(g, w4)


def kernel(features, adjs, W1, att_src1, att_dst1, W4):
    src = adjs[0]
    dst = adjs[1]
    pad = E_PAD - E
    srcp = jnp.concatenate(
        [src, jnp.zeros((pad,), jnp.int32)]).reshape(NW, CPT, CHUNK)
    dstp = jnp.concatenate(
        [dst, jnp.full((pad,), N, jnp.int32)]).reshape(NW, CPT, CHUNK)

    xpad = jnp.concatenate(
        [features, jnp.zeros((N_ACC - N, IN_DIM), jnp.float32)])
    h, asrc, adst = _tc_proj(xpad, W1,
                             att_src1.reshape(OUT_DIM, 1),
                             att_dst1.reshape(OUT_DIM, 1))
    num, den = _sc_gat(h, asrc.reshape(N_ACC), adst.reshape(N_ACC),
                       srcp, dstp)
    h1p = _tc_combine(num, den)
    g = _sc_agg(h1p, srcp, dstp)
    h4 = _tc_out(g, W4)
    return (h1p[:N], h4)


# speedup vs baseline: 1.7802x; 1.0262x over previous
import jax
import jax.numpy as jnp
from jax import lax
from jax.experimental import pallas as pl
from jax.experimental.pallas import tpu as pltpu
from jax.experimental.pallas import tpu_sc as plsc

N = 10000
E = 320000
IN_DIM = 128
OUT_DIM = 64

NC = 2
NS = 16
LANES = 16
NW = NC * NS

CHUNK = 128
CPT = (E + NW * CHUNK - 1) // (NW * CHUNK)
E_PAD = NW * CPT * CHUNK
ROWS_PT = 640
N_ACC = NS * ROWS_PT


def _mesh():
    return plsc.VectorSubcoreMesh(
        core_axis_name="c", subcore_axis_name="s",
        num_cores=NC, num_subcores=NS)


_SC_PARAMS = pltpu.CompilerParams(
    needs_layout_passes=False, use_tc_tiling_on_sc=False)


def _sc_gat_body(h_hbm, asrc_hbm, adst_hbm, src_hbm, dst_hbm,
                 num_out, den_out,
                 src_v, dst_v, rows_a, as_a, ad_a, exv, zrow_v, zden_v,
                 h_sh, asrc_sh, adst_sh, num_sh, den_sh):
    c = lax.axis_index("c")
    s = lax.axis_index("s")
    wid = c * NS + s
    zero16 = jnp.zeros((LANES,), jnp.float32)
    for r in range(CHUNK):
        for d in range(OUT_DIM // LANES):
            zrow_v[r, pl.ds(d * LANES, LANES)] = zero16
    for i in range(CHUNK // LANES):
        zden_v[pl.ds(i * LANES, LANES)] = zero16
    base = s * ROWS_PT
    for k in range(ROWS_PT // CHUNK):
        pltpu.sync_copy(zrow_v, num_sh.at[pl.ds(base + k * CHUNK, CHUNK)])
        pltpu.sync_copy(zden_v, den_sh.at[pl.ds(base + k * CHUNK, CHUNK)])
    pltpu.sync_copy(h_hbm.at[pl.ds(base, ROWS_PT)],
                    h_sh.at[pl.ds(base, ROWS_PT)])
    pltpu.sync_copy(asrc_hbm.at[pl.ds(base, ROWS_PT)],
                    asrc_sh.at[pl.ds(base, ROWS_PT)])
    pltpu.sync_copy(adst_hbm.at[pl.ds(base, ROWS_PT)],
                    adst_sh.at[pl.ds(base, ROWS_PT)])
    plsc.subcore_barrier()

    pltpu.sync_copy(src_hbm.at[wid], src_v)
    pltpu.sync_copy(dst_hbm.at[wid], dst_v)

    @pl.loop(0, CPT)
    def _chunk(j):
        sidx = src_v.at[j]
        didx = dst_v.at[j]
        pltpu.sync_copy(asrc_sh.at[sidx], as_a)
        pltpu.sync_copy(adst_sh.at[didx], ad_a)
        pltpu.sync_copy(h_sh.at[sidx], rows_a)
        for i in range(CHUNK // LANES):
            sl = pl.ds(i * LANES, LANES)
            t = as_a[sl] + ad_a[sl]
            t = jnp.where(t >= 0.0, t, 0.2 * t)
            exv[sl] = jnp.exp(t)
        zero = jnp.zeros((LANES,), jnp.int32)

        @pl.loop(0, CHUNK, unroll=4)
        def _scale(e):
            b = plsc.load_gather(exv, [zero + e])
            for d in range(OUT_DIM // LANES):
                sl = pl.ds(d * LANES, LANES)
                rows_a[e, sl] = rows_a[e, sl] * b

        pltpu.sync_copy(exv, den_sh.at[didx], add=True)
        pltpu.sync_copy(rows_a, num_sh.at[didx], add=True)

    plsc.subcore_barrier()
    pltpu.sync_copy(num_sh.at[pl.ds(base, ROWS_PT)],
                    num_out.at[c, pl.ds(base, ROWS_PT)])
    pltpu.sync_copy(den_sh.at[pl.ds(base, ROWS_PT)],
                    den_out.at[c, pl.ds(base, ROWS_PT)])


def _sc_gat(h, asrc, adst, srcp, dstp):
    f = pl.kernel(
        _sc_gat_body,
        out_type=(jax.ShapeDtypeStruct((NC, N_ACC, OUT_DIM), jnp.float32),
                  jax.ShapeDtypeStruct((NC, N_ACC), jnp.float32)),
        mesh=_mesh(),
        compiler_params=_SC_PARAMS,
        scratch_types=[
            pltpu.VMEM((CPT, CHUNK), jnp.int32),
            pltpu.VMEM((CPT, CHUNK), jnp.int32),
            pltpu.VMEM((CHUNK, OUT_DIM), jnp.float32),
            pltpu.VMEM((CHUNK,), jnp.float32),
            pltpu.VMEM((CHUNK,), jnp.float32),
            pltpu.VMEM((CHUNK,), jnp.float32),
            pltpu.VMEM((CHUNK, OUT_DIM), jnp.float32),
            pltpu.VMEM((CHUNK,), jnp.float32),
            pltpu.VMEM_SHARED((N_ACC, OUT_DIM), jnp.float32),
            pltpu.VMEM_SHARED((N_ACC,), jnp.float32),
            pltpu.VMEM_SHARED((N_ACC,), jnp.float32),
            pltpu.VMEM_SHARED((N_ACC, OUT_DIM), jnp.float32),
            pltpu.VMEM_SHARED((N_ACC,), jnp.float32),
        ],
    )
    return f(h, asrc, adst, srcp, dstp)


def _sc_agg_body(h1_hbm, src_hbm, dst_hbm, g_out,
                 src_v, dst_v, r0, zrow_v, h1_sh, g_sh):
    c = lax.axis_index("c")
    s = lax.axis_index("s")
    wid = c * NS + s
    zero16 = jnp.zeros((LANES,), jnp.float32)
    for r in range(CHUNK):
        for d in range(OUT_DIM // LANES):
            zrow_v[r, pl.ds(d * LANES, LANES)] = zero16
    base = s * ROWS_PT
    for k in range(ROWS_PT // CHUNK):
        pltpu.sync_copy(zrow_v, g_sh.at[pl.ds(base + k * CHUNK, CHUNK)])
    pltpu.sync_copy(h1_hbm.at[pl.ds(base, ROWS_PT)],
                    h1_sh.at[pl.ds(base, ROWS_PT)])
    plsc.subcore_barrier()

    pltpu.sync_copy(src_hbm.at[wid], src_v)
    pltpu.sync_copy(dst_hbm.at[wid], dst_v)

    @pl.loop(0, CPT)
    def _chunk(j):
        pltpu.sync_copy(h1_sh.at[src_v.at[j]], r0)
        pltpu.sync_copy(r0, g_sh.at[dst_v.at[j]], add=True)

    plsc.subcore_barrier()
    pltpu.sync_copy(g_sh.at[pl.ds(base, ROWS_PT)],
                    g_out.at[c, pl.ds(base, ROWS_PT)])


def _sc_agg(h1, srcp, dstp):
    f = pl.kernel(
        _sc_agg_body,
        out_type=jax.ShapeDtypeStruct((NC, N_ACC, OUT_DIM), jnp.float32),
        mesh=_mesh(),
        compiler_params=_SC_PARAMS,
        scratch_types=[
            pltpu.VMEM((CPT, CHUNK), jnp.int32),
            pltpu.VMEM((CPT, CHUNK), jnp.int32),
            pltpu.VMEM((CHUNK, OUT_DIM), jnp.float32),
            pltpu.VMEM((CHUNK, OUT_DIM), jnp.float32),
            pltpu.VMEM_SHARED((N_ACC, OUT_DIM), jnp.float32),
            pltpu.VMEM_SHARED((N_ACC, OUT_DIM), jnp.float32),
        ],
    )
    return f(h1, srcp, dstp)


_BLK = 2048


def _tc_proj_body(x_ref, w_ref, as_ref, ad_ref, h_ref, asrc_ref, adst_ref):
    h = jnp.dot(x_ref[...], w_ref[...], preferred_element_type=jnp.float32)
    h_ref[...] = h
    asrc_ref[...] = jnp.dot(h, as_ref[...], preferred_element_type=jnp.float32)
    adst_ref[...] = jnp.dot(h, ad_ref[...], preferred_element_type=jnp.float32)


def _tc_proj(x, w1, att_s, att_d):
    return pl.pallas_call(
        _tc_proj_body,
        grid=(N_ACC // _BLK,),
        in_specs=[
            pl.BlockSpec((_BLK, IN_DIM), lambda i: (i, 0)),
            pl.BlockSpec((IN_DIM, OUT_DIM), lambda i: (0, 0)),
            pl.BlockSpec((OUT_DIM, 1), lambda i: (0, 0)),
            pl.BlockSpec((OUT_DIM, 1), lambda i: (0, 0)),
        ],
        out_specs=[
            pl.BlockSpec((_BLK, OUT_DIM), lambda i: (i, 0)),
            pl.BlockSpec((_BLK, 1), lambda i: (i, 0)),
            pl.BlockSpec((_BLK, 1), lambda i: (i, 0)),
        ],
        out_shape=[
            jax.ShapeDtypeStruct((N_ACC, OUT_DIM), jnp.float32),
            jax.ShapeDtypeStruct((N_ACC, 1), jnp.float32),
            jax.ShapeDtypeStruct((N_ACC, 1), jnp.float32),
        ],
    )(x, w1, att_s, att_d)


def _tc_combine_body(num_ref, den_ref, h1_ref):
    n = num_ref[0] + num_ref[1]
    d = den_ref[0] + den_ref[1] + 1e-16
    o = n / d
    h1_ref[...] = jnp.where(o > 0.0, o, jnp.exp(jnp.minimum(o, 0.0)) - 1.0)


def _tc_combine(num, den):
    den = den.reshape(NC, N_ACC, 1)
    return pl.pallas_call(
        _tc_combine_body,
        grid=(N_ACC // _BLK,),
        in_specs=[
            pl.BlockSpec((NC, _BLK, OUT_DIM), lambda i: (0, i, 0)),
            pl.BlockSpec((NC, _BLK, 1), lambda i: (0, i, 0)),
        ],
        out_specs=pl.BlockSpec((_BLK, OUT_DIM), lambda i: (i, 0)),
        out_shape=jax.ShapeDtypeStruct((N_ACC, OUT_DIM), jnp.float32),
    )(num, den)


def _tc_out_body(g_ref, w_ref, h4_ref):
    g = g_ref[0] + g_ref[1]
    h4_ref[...] = jnp.dot(g, w_ref[...], preferred_element_type=jnp.float32)


def _tc_out(g, w4):
    return pl.pallas_call(
        _tc_out_body,
        grid=(N // 2000,),
        in_specs=[
            pl.BlockSpec((NC, 2000, OUT_DIM), lambda i: (0, i, 0)),
            pl.BlockSpec((OUT_DIM, IN_DIM), lambda i: (0, 0)),
        ],
        out_specs=pl.BlockSpec((2000, IN_DIM), lambda i: (i, 0)),
        out_shape=jax.ShapeDtypeStruct((N, IN_DIM), jnp.float32),
    )(g, w4)


def kernel(features, adjs, W1, att_src1, att_dst1, W4):
    src = adjs[0]
    dst = adjs[1]
    pad = E_PAD - E
    srcp = jnp.concatenate(
        [src, jnp.zeros((pad,), jnp.int32)]).reshape(NW, CPT, CHUNK)
    dstp = jnp.concatenate(
        [dst, jnp.full((pad,), N, jnp.int32)]).reshape(NW, CPT, CHUNK)

    xpad = jnp.concatenate(
        [features, jnp.zeros((N_ACC - N, IN_DIM), jnp.float32)])
    h, asrc, adst = _tc_proj(xpad, W1,
                             att_src1.reshape(OUT_DIM, 1),
                             att_dst1.reshape(OUT_DIM, 1))
    num, den = _sc_gat(h, asrc.reshape(N_ACC), adst.reshape(N_ACC),
                       srcp, dstp)
    h1p = _tc_combine(num, den)
    g = _sc_agg(h1p, srcp, dstp)
    h4 = _tc_out(g, W4)
    return (h1p[:N], h4)
